# Initial kernel scaffold; baseline (speedup 1.0000x reference)
import jax
import jax.numpy as jnp
from jax.experimental import pallas as pl


def kernel(node_type, c, gm, pos, r, vid, edge_index, batch, W1, b1, W2, b2, Wg0, bg0, Wg1, bg1, Wg2, bg2, gam0, bet0, gam1, bet1, Wo, bo):
    raise NotImplementedError("write your pallas kernel here")


# speedup vs baseline: 11.8677x; 11.8677x over previous
import functools

import jax
import jax.numpy as jnp
from jax import lax
from jax.experimental import pallas as pl
from jax.experimental.pallas import tpu as pltpu
from jax.experimental.pallas import tpu_sc as plsc

_N = 10000
_E = 640000
_H = 128
_NT = 16
_G = 64

_NC = 2
_NS = 16
_NW = _NC * _NS
_CH = 80
_EPT = _E // _NW
_RPT = _EPT // _CH
_STRIPE = 624
_TAIL = _N - _NS * _STRIPE
_TAIL_OFF = _NS * _STRIPE

_MESH = plsc.VectorSubcoreMesh(core_axis_name="c", subcore_axis_name="s")


def _deg_body(dstr_hbm, ones_hbm, out_hbm, acc_sh, idx_v, ones_v):
    cid = lax.axis_index("c")
    sid = lax.axis_index("s")
    wid = sid * _NC + cid
    pltpu.sync_copy(ones_hbm.at[pl.ds(sid * _STRIPE, _STRIPE)],
                    acc_sh.at[pl.ds(sid * _STRIPE, _STRIPE)])

    @pl.when(sid == _NS - 1)
    def _():
        pltpu.sync_copy(ones_hbm.at[pl.ds(_TAIL_OFF, _TAIL)],
                        acc_sh.at[pl.ds(_TAIL_OFF, _TAIL)])

    pltpu.sync_copy(ones_hbm.at[pl.ds(0, _CH)], ones_v)
    plsc.subcore_barrier()
    base = wid * _EPT

    def body(j, carry):
        pltpu.sync_copy(dstr_hbm.at[pl.ds(base + j * _CH, _CH)], idx_v)
        pltpu.sync_copy(ones_v, acc_sh.at[idx_v], add=True)
        return carry

    lax.fori_loop(0, _RPT, body, 0)
    plsc.subcore_barrier()
    pltpu.sync_copy(acc_sh.at[pl.ds(sid * _STRIPE, _STRIPE)],
                    out_hbm.at[cid, pl.ds(sid * _STRIPE, _STRIPE)])

    @pl.when(sid == _NS - 1)
    def _():
        pltpu.sync_copy(acc_sh.at[pl.ds(_TAIL_OFF, _TAIL)],
                        out_hbm.at[cid, pl.ds(_TAIL_OFF, _TAIL)])


_deg = pl.kernel(
    _deg_body,
    out_type=jax.ShapeDtypeStruct((_NC, _N, 16), jnp.float32),
    mesh=_MESH,
    scratch_types=[
        pltpu.VMEM_SHARED((_N, 16), jnp.float32),
        pltpu.VMEM((_CH,), jnp.int32),
        pltpu.VMEM((_CH, 16), jnp.float32),
    ],
)


def _mp_body(u_hbm, srcr_hbm, dstr_hbm, out_hbm, acc_sh, sidx_v, didx_v,
             rows_v, sem):
    cid = lax.axis_index("c")
    sid = lax.axis_index("s")
    wid = sid * _NC + cid
    pltpu.sync_copy(u_hbm.at[pl.ds(sid * _STRIPE, _STRIPE)],
                    acc_sh.at[pl.ds(sid * _STRIPE, _STRIPE)])

    @pl.when(sid == _NS - 1)
    def _():
        pltpu.sync_copy(u_hbm.at[pl.ds(_TAIL_OFF, _TAIL)],
                        acc_sh.at[pl.ds(_TAIL_OFF, _TAIL)])

    plsc.subcore_barrier()
    base = wid * _EPT

    def body(j, carry):
        pltpu.sync_copy(srcr_hbm.at[pl.ds(base + j * _CH, _CH)], sidx_v)
        pltpu.sync_copy(dstr_hbm.at[pl.ds(base + j * _CH, _CH)], didx_v)
        pltpu.async_copy(u_hbm.at[sidx_v], rows_v, sem).wait()
        pltpu.sync_copy(rows_v, acc_sh.at[didx_v], add=True)
        return carry

    lax.fori_loop(0, _RPT, body, 0)
    plsc.subcore_barrier()
    pltpu.sync_copy(acc_sh.at[pl.ds(sid * _STRIPE, _STRIPE)],
                    out_hbm.at[cid, pl.ds(sid * _STRIPE, _STRIPE)])

    @pl.when(sid == _NS - 1)
    def _():
        pltpu.sync_copy(acc_sh.at[pl.ds(_TAIL_OFF, _TAIL)],
                        out_hbm.at[cid, pl.ds(_TAIL_OFF, _TAIL)])


_mp = pl.kernel(
    _mp_body,
    out_type=jax.ShapeDtypeStruct((_NC, _N, _H), jnp.float32),
    mesh=_MESH,
    scratch_types=[
        pltpu.VMEM_SHARED((_N, _H), jnp.float32),
        pltpu.VMEM((_CH,), jnp.int32),
        pltpu.VMEM((_CH,), jnp.int32),
        pltpu.VMEM((_CH, _H), jnp.float32),
        pltpu.SemaphoreType.DMA,
    ],
)


def _emb_body(nt_ref, feats_ref, degp_ref, w1_ref, b1_ref, w2_ref, b2_ref,
              wg0_ref, u0_ref, dinv_ref):
    oh = (nt_ref[:] == lax.broadcasted_iota(jnp.int32, (1, _NT), 1))
    x_t = jnp.dot(oh.astype(jnp.float32), w1_ref[:],
                  preferred_element_type=jnp.float32) + b1_ref[:]
    x_num = jnp.dot(feats_ref[:], w2_ref[:],
                    preferred_element_type=jnp.float32) + b2_ref[:]
    deg = degp_ref[0, :, 0:1] + degp_ref[1, :, 0:1] - 1.0
    dinv = lax.rsqrt(deg)
    z = jnp.concatenate([x_t, x_num], axis=1)
    h0 = jnp.dot(z, wg0_ref[:], preferred_element_type=jnp.float32)
    u0_ref[:] = h0 * dinv
    dinv_ref[:] = dinv


_emb = pl.pallas_call(
    _emb_body,
    out_shape=[
        jax.ShapeDtypeStruct((_N, _H), jnp.float32),
        jax.ShapeDtypeStruct((_N, 1), jnp.float32),
    ],
)


def _comb_body(accp_ref, u_ref, dinv_ref, b_ref, gam_ref, bet_ref, w_ref,
               out_ref):
    s = accp_ref[0] + accp_ref[1] - u_ref[:]
    t = s * dinv_ref[:] + b_ref[:]
    mu = jnp.mean(t, axis=0, keepdims=True)
    var = jnp.mean((t - mu) * (t - mu), axis=0, keepdims=True)
    zn = (t - mu) * lax.rsqrt(var + 1e-5) * gam_ref[:] + bet_ref[:]
    z = jnp.maximum(zn, 0.0)
    out_ref[:] = jnp.dot(z, w_ref[:],
                         preferred_element_type=jnp.float32) * dinv_ref[:]


_comb = pl.pallas_call(
    _comb_body,
    out_shape=jax.ShapeDtypeStruct((_N, _H), jnp.float32),
)


def _final_body(accp_ref, u_ref, dinv_ref, b_ref, batch_ref, wo_ref, bo_ref,
                out_ref):
    s = accp_ref[0] + accp_ref[1] - u_ref[:]
    t = s * dinv_ref[:] + b_ref[:]
    z = jnp.maximum(t, 0.0)
    ohb = (batch_ref[:] == lax.broadcasted_iota(jnp.int32, (1, _G), 1))
    zz = jnp.concatenate([z, jnp.ones((_N, 1), jnp.float32)], axis=1)
    s2 = lax.dot_general(ohb.astype(jnp.float32), zz,
                         (((0,), (0,)), ((), ())),
                         preferred_element_type=jnp.float32)
    pooled = s2[:, :_H] / jnp.maximum(s2[:, _H:_H + 1], 1.0)
    out_ref[:] = jnp.dot(pooled, wo_ref[:],
                         preferred_element_type=jnp.float32) + bo_ref[:]


_final = pl.pallas_call(
    _final_body,
    out_shape=jax.ShapeDtypeStruct((_G, 4), jnp.float32),
)


def kernel(node_type, c, gm, pos, r, vid, edge_index, batch, W1, b1, W2, b2,
           Wg0, bg0, Wg1, bg1, Wg2, bg2, gam0, bet0, gam1, bet1, Wo, bo):
    nt2 = node_type.reshape(_N, 1).astype(jnp.int32)
    feats = jnp.stack([c, gm, pos, r, vid], axis=-1)
    srcr = edge_index[0].astype(jnp.int32).reshape(_E)
    dstr = edge_index[1].astype(jnp.int32).reshape(_E)
    batch2 = batch.reshape(_N, 1).astype(jnp.int32)
    ones16 = jnp.ones((_N, 16), jnp.float32)

    degp = _deg(dstr, ones16)
    u0, dinv = _emb(nt2, feats, degp, W1, b1.reshape(1, _H), W2,
                    b2.reshape(1, _H), Wg0)
    acc0 = _mp(u0, srcr, dstr)
    u1 = _comb(acc0, u0, dinv, bg0.reshape(1, _H), gam0.reshape(1, _H),
               bet0.reshape(1, _H), Wg1)
    acc1 = _mp(u1, srcr, dstr)
    u2 = _comb(acc1, u1, dinv, bg1.reshape(1, _H), gam1.reshape(1, _H),
               bet1.reshape(1, _H), Wg2)
    acc2 = _mp(u2, srcr, dstr)
    return _final(acc2, u2, dinv, bg2.reshape(1, _H), batch2, Wo,
                  bo.reshape(1, 4))


# speedup vs baseline: 29.4810x; 2.4841x over previous
import functools

import jax
import jax.numpy as jnp
from jax import lax
from jax.experimental import pallas as pl
from jax.experimental.pallas import tpu as pltpu
from jax.experimental.pallas import tpu_sc as plsc

_N = 10000
_E = 640000
_H = 128
_NT = 16
_G = 64

_NC = 2
_NS = 16
_NW = _NC * _NS
_CH = 80
_EPT = _E // _NW
_RPT = _EPT // _CH
_HALF = _EPT // 2
_CPS = _HALF // _CH
_STRIPE = 624
_TAIL = _N - _NS * _STRIPE
_TAIL_OFF = _NS * _STRIPE

_MESH = plsc.VectorSubcoreMesh(core_axis_name="c", subcore_axis_name="s")


def _deg_body(dstr_hbm, ones_hbm, out_hbm, acc_sh, didx_st, idx_v, ones_v):
    cid = lax.axis_index("c")
    sid = lax.axis_index("s")
    wid = sid * _NC + cid
    pltpu.sync_copy(ones_hbm.at[pl.ds(sid * _STRIPE, _STRIPE)],
                    acc_sh.at[pl.ds(sid * _STRIPE, _STRIPE)])

    @pl.when(sid == _NS - 1)
    def _():
        pltpu.sync_copy(ones_hbm.at[pl.ds(_TAIL_OFF, _TAIL)],
                        acc_sh.at[pl.ds(_TAIL_OFF, _TAIL)])

    pltpu.sync_copy(ones_hbm.at[pl.ds(0, _CH)], ones_v)
    plsc.subcore_barrier()

    def _cpidx(c, db):
        for k in range(_CH // 16):
            db[pl.ds(16 * k, 16)] = didx_st[pl.ds(c * _CH + 16 * k, 16)]

    for s in range(2):
        sbase = wid * _EPT + s * _HALF
        pltpu.sync_copy(dstr_hbm.at[pl.ds(sbase, _HALF)], didx_st)

        def body(j, carry):
            _cpidx(j, idx_v)
            pltpu.sync_copy(ones_v, acc_sh.at[idx_v], add=True)
            return carry

        lax.fori_loop(0, _CPS, body, 0)

    plsc.subcore_barrier()
    pltpu.sync_copy(acc_sh.at[pl.ds(sid * _STRIPE, _STRIPE)],
                    out_hbm.at[cid, pl.ds(sid * _STRIPE, _STRIPE)])

    @pl.when(sid == _NS - 1)
    def _():
        pltpu.sync_copy(acc_sh.at[pl.ds(_TAIL_OFF, _TAIL)],
                        out_hbm.at[cid, pl.ds(_TAIL_OFF, _TAIL)])


_deg = pl.kernel(
    _deg_body,
    out_type=jax.ShapeDtypeStruct((_NC, _N, 16), jnp.float32),
    mesh=_MESH,
    scratch_types=[
        pltpu.VMEM_SHARED((_N, 16), jnp.float32),
        pltpu.VMEM((_HALF,), jnp.int32),
        pltpu.VMEM((_CH,), jnp.int32),
        pltpu.VMEM((_CH, 16), jnp.float32),
    ],
)


def _mp_body(u_hbm, srcr_hbm, dstr_hbm, out_hbm, acc_sh, sidx_st, didx_st,
             db0, db1, rows0, rows1, sem0, sem1):
    cid = lax.axis_index("c")
    sid = lax.axis_index("s")
    wid = sid * _NC + cid
    pltpu.sync_copy(u_hbm.at[pl.ds(sid * _STRIPE, _STRIPE)],
                    acc_sh.at[pl.ds(sid * _STRIPE, _STRIPE)])

    @pl.when(sid == _NS - 1)
    def _():
        pltpu.sync_copy(u_hbm.at[pl.ds(_TAIL_OFF, _TAIL)],
                        acc_sh.at[pl.ds(_TAIL_OFF, _TAIL)])

    plsc.subcore_barrier()

    def _gather(c, rbuf, sem_):
        pltpu.async_copy(u_hbm.at[sidx_st.at[pl.ds(c * _CH, _CH)]],
                         rbuf, sem_)

    def _gwait(rbuf, sem_):
        pltpu.make_async_copy(u_hbm.at[sidx_st.at[pl.ds(0, _CH)]],
                              rbuf, sem_).wait()

    def _cpidx(c, db):
        for k in range(_CH // 16):
            db[pl.ds(16 * k, 16)] = didx_st[pl.ds(c * _CH + 16 * k, 16)]

    def _scat(rbuf, db):
        pltpu.sync_copy(rbuf, acc_sh.at[db], add=True)

    for s in range(2):
        sbase = wid * _EPT + s * _HALF
        pltpu.sync_copy(srcr_hbm.at[pl.ds(sbase, _HALF)], sidx_st)
        pltpu.sync_copy(dstr_hbm.at[pl.ds(sbase, _HALF)], didx_st)
        _cpidx(0, db0)
        _gather(0, rows0, sem0)

        def pair(i, carry):
            b = 2 * i + 1
            _cpidx(b, db1)
            _gather(b, rows1, sem1)
            _gwait(rows0, sem0)
            _scat(rows0, db0)
            _cpidx(b + 1, db0)
            _gather(b + 1, rows0, sem0)
            _gwait(rows1, sem1)
            _scat(rows1, db1)
            return carry

        lax.fori_loop(0, (_CPS - 1) // 2, pair, 0)
        _gwait(rows0, sem0)
        _scat(rows0, db0)

    plsc.subcore_barrier()
    pltpu.sync_copy(acc_sh.at[pl.ds(sid * _STRIPE, _STRIPE)],
                    out_hbm.at[cid, pl.ds(sid * _STRIPE, _STRIPE)])

    @pl.when(sid == _NS - 1)
    def _():
        pltpu.sync_copy(acc_sh.at[pl.ds(_TAIL_OFF, _TAIL)],
                        out_hbm.at[cid, pl.ds(_TAIL_OFF, _TAIL)])


_mp = pl.kernel(
    _mp_body,
    out_type=jax.ShapeDtypeStruct((_NC, _N, _H), jnp.float32),
    mesh=_MESH,
    scratch_types=[
        pltpu.VMEM_SHARED((_N, _H), jnp.float32),
        pltpu.VMEM((_HALF,), jnp.int32),
        pltpu.VMEM((_HALF,), jnp.int32),
        pltpu.VMEM((_CH,), jnp.int32),
        pltpu.VMEM((_CH,), jnp.int32),
        pltpu.VMEM((_CH, _H), jnp.float32),
        pltpu.VMEM((_CH, _H), jnp.float32),
        pltpu.SemaphoreType.DMA,
        pltpu.SemaphoreType.DMA,
    ],
)


def _emb_body(nt_ref, feats_ref, degp_ref, w1_ref, b1_ref, w2_ref, b2_ref,
              wg0_ref, u0_ref, dinv_ref):
    oh = (nt_ref[:] == lax.broadcasted_iota(jnp.int32, (1, _NT), 1))
    x_t = jnp.dot(oh.astype(jnp.float32), w1_ref[:],
                  preferred_element_type=jnp.float32) + b1_ref[:]
    x_num = jnp.dot(feats_ref[:], w2_ref[:],
                    preferred_element_type=jnp.float32) + b2_ref[:]
    deg = degp_ref[0, :, 0:1] + degp_ref[1, :, 0:1] - 1.0
    dinv = lax.rsqrt(deg)
    z = jnp.concatenate([x_t, x_num], axis=1)
    h0 = jnp.dot(z, wg0_ref[:], preferred_element_type=jnp.float32)
    u0_ref[:] = h0 * dinv
    dinv_ref[:] = dinv


_emb = pl.pallas_call(
    _emb_body,
    out_shape=[
        jax.ShapeDtypeStruct((_N, _H), jnp.float32),
        jax.ShapeDtypeStruct((_N, 1), jnp.float32),
    ],
)


def _comb_body(accp_ref, u_ref, dinv_ref, b_ref, gam_ref, bet_ref, w_ref,
               out_ref):
    s = accp_ref[0] + accp_ref[1] - u_ref[:]
    t = s * dinv_ref[:] + b_ref[:]
    mu = jnp.mean(t, axis=0, keepdims=True)
    var = jnp.mean((t - mu) * (t - mu), axis=0, keepdims=True)
    zn = (t - mu) * lax.rsqrt(var + 1e-5) * gam_ref[:] + bet_ref[:]
    z = jnp.maximum(zn, 0.0)
    out_ref[:] = jnp.dot(z, w_ref[:],
                         preferred_element_type=jnp.float32) * dinv_ref[:]


_comb = pl.pallas_call(
    _comb_body,
    out_shape=jax.ShapeDtypeStruct((_N, _H), jnp.float32),
)


def _final_body(accp_ref, u_ref, dinv_ref, b_ref, batch_ref, wo_ref, bo_ref,
                out_ref):
    s = accp_ref[0] + accp_ref[1] - u_ref[:]
    t = s * dinv_ref[:] + b_ref[:]
    z = jnp.maximum(t, 0.0)
    ohb = (batch_ref[:] == lax.broadcasted_iota(jnp.int32, (1, _G), 1))
    zz = jnp.concatenate([z, jnp.ones((_N, 1), jnp.float32)], axis=1)
    s2 = lax.dot_general(ohb.astype(jnp.float32), zz,
                         (((0,), (0,)), ((), ())),
                         preferred_element_type=jnp.float32)
    pooled = s2[:, :_H] / jnp.maximum(s2[:, _H:_H + 1], 1.0)
    out_ref[:] = jnp.dot(pooled, wo_ref[:],
                         preferred_element_type=jnp.float32) + bo_ref[:]


_final = pl.pallas_call(
    _final_body,
    out_shape=jax.ShapeDtypeStruct((_G, 4), jnp.float32),
)


def kernel(node_type, c, gm, pos, r, vid, edge_index, batch, W1, b1, W2, b2,
           Wg0, bg0, Wg1, bg1, Wg2, bg2, gam0, bet0, gam1, bet1, Wo, bo):
    nt2 = node_type.reshape(_N, 1).astype(jnp.int32)
    feats = jnp.stack([c, gm, pos, r, vid], axis=-1)
    srcr = edge_index[0].astype(jnp.int32).reshape(_E)
    dstr = edge_index[1].astype(jnp.int32).reshape(_E)
    batch2 = batch.reshape(_N, 1).astype(jnp.int32)
    ones16 = jnp.ones((_N, 16), jnp.float32)

    degp = _deg(dstr, ones16)
    u0, dinv = _emb(nt2, feats, degp, W1, b1.reshape(1, _H), W2,
                    b2.reshape(1, _H), Wg0)
    acc0 = _mp(u0, srcr, dstr)
    u1 = _comb(acc0, u0, dinv, bg0.reshape(1, _H), gam0.reshape(1, _H),
               bet0.reshape(1, _H), Wg1)
    acc1 = _mp(u1, srcr, dstr)
    u2 = _comb(acc1, u1, dinv, bg1.reshape(1, _H), gam1.reshape(1, _H),
               bet1.reshape(1, _H), Wg2)
    acc2 = _mp(u2, srcr, dstr)
    return _final(acc2, u2, dinv, bg2.reshape(1, _H), batch2, Wo,
                  bo.reshape(1, 4))


# speedup vs baseline: 34.0721x; 1.1557x over previous
import functools

import jax
import jax.numpy as jnp
from jax import lax
from jax.experimental import pallas as pl
from jax.experimental.pallas import tpu as pltpu
from jax.experimental.pallas import tpu_sc as plsc

_N = 10000
_E = 640000
_H = 128
_NT = 16
_G = 64

_NC = 2
_NS = 16
_NW = _NC * _NS
_CH = 80
_EPT = _E // _NW
_RPT = _EPT // _CH
_HALF = _EPT // 2
_CPS = _HALF // _CH
_MST = 4000
_MCPS = _MST // _CH
_MNST = _EPT // _MST
_STRIPE = 624
_TAIL = _N - _NS * _STRIPE
_TAIL_OFF = _NS * _STRIPE

_MESH = plsc.VectorSubcoreMesh(core_axis_name="c", subcore_axis_name="s")


def _deg_body(dstr_hbm, ones_hbm, out_hbm, acc_sh, didx_st, idx_v, ones_v):
    cid = lax.axis_index("c")
    sid = lax.axis_index("s")
    wid = sid * _NC + cid
    pltpu.sync_copy(ones_hbm.at[pl.ds(sid * _STRIPE, _STRIPE)],
                    acc_sh.at[pl.ds(sid * _STRIPE, _STRIPE)])

    @pl.when(sid == _NS - 1)
    def _():
        pltpu.sync_copy(ones_hbm.at[pl.ds(_TAIL_OFF, _TAIL)],
                        acc_sh.at[pl.ds(_TAIL_OFF, _TAIL)])

    pltpu.sync_copy(ones_hbm.at[pl.ds(0, _CH)], ones_v)
    plsc.subcore_barrier()

    def _cpidx(c, db):
        for k in range(_CH // 16):
            db[pl.ds(16 * k, 16)] = didx_st[pl.ds(c * _CH + 16 * k, 16)]

    for s in range(2):
        sbase = wid * _EPT + s * _HALF
        pltpu.sync_copy(dstr_hbm.at[pl.ds(sbase, _HALF)], didx_st)

        def body(j, carry):
            _cpidx(j, idx_v)
            pltpu.sync_copy(ones_v, acc_sh.at[idx_v], add=True)
            return carry

        lax.fori_loop(0, _CPS, body, 0)

    plsc.subcore_barrier()
    pltpu.sync_copy(acc_sh.at[pl.ds(sid * _STRIPE, _STRIPE)],
                    out_hbm.at[cid, pl.ds(sid * _STRIPE, _STRIPE)])

    @pl.when(sid == _NS - 1)
    def _():
        pltpu.sync_copy(acc_sh.at[pl.ds(_TAIL_OFF, _TAIL)],
                        out_hbm.at[cid, pl.ds(_TAIL_OFF, _TAIL)])


_deg = pl.kernel(
    _deg_body,
    out_type=jax.ShapeDtypeStruct((_NC, _N, 16), jnp.float32),
    mesh=_MESH,
    scratch_types=[
        pltpu.VMEM_SHARED((_N, 16), jnp.float32),
        pltpu.VMEM((_HALF,), jnp.int32),
        pltpu.VMEM((_CH,), jnp.int32),
        pltpu.VMEM((_CH, 16), jnp.float32),
    ],
)


def _mp_body(u_hbm, srcr_hbm, dstr_hbm, out_hbm, acc_sh, sidx_st, didx_st,
             db0, db1, db2, rows0, rows1, rows2, g0, g1, g2, ss0, ss1, ss2):
    cid = lax.axis_index("c")
    sid = lax.axis_index("s")
    wid = sid * _NC + cid
    pltpu.sync_copy(u_hbm.at[pl.ds(sid * _STRIPE, _STRIPE)],
                    acc_sh.at[pl.ds(sid * _STRIPE, _STRIPE)])

    @pl.when(sid == _NS - 1)
    def _():
        pltpu.sync_copy(u_hbm.at[pl.ds(_TAIL_OFF, _TAIL)],
                        acc_sh.at[pl.ds(_TAIL_OFF, _TAIL)])

    plsc.subcore_barrier()

    dbs = (db0, db1, db2)
    rows = (rows0, rows1, rows2)
    gs = (g0, g1, g2)
    sss = (ss0, ss1, ss2)

    def _gather(c, k):
        pltpu.async_copy(u_hbm.at[sidx_st.at[pl.ds(c * _CH, _CH)]],
                         rows[k], gs[k])

    def _gwait(k):
        pltpu.make_async_copy(u_hbm.at[sidx_st.at[pl.ds(0, _CH)]],
                              rows[k], gs[k]).wait()

    def _cpidx(c, k):
        for q in range(_CH // 16):
            dbs[k][pl.ds(16 * q, 16)] = didx_st[pl.ds(c * _CH + 16 * q, 16)]

    def _scat(k):
        pltpu.async_copy(rows[k], acc_sh.at[dbs[k]], sss[k], add=True)

    def _swait(k):
        pltpu.make_async_copy(rows[k], acc_sh.at[dbs[k]], sss[k]).wait()

    for s in range(_MNST):
        sbase = wid * _EPT + s * _MST
        pltpu.sync_copy(srcr_hbm.at[pl.ds(sbase, _MST)], sidx_st)
        pltpu.sync_copy(dstr_hbm.at[pl.ds(sbase, _MST)], didx_st)
        _cpidx(0, 0)
        _gather(0, 0)
        _cpidx(1, 1)
        _gather(1, 1)
        _cpidx(2, 2)
        _gather(2, 2)
        _gwait(0)
        _scat(0)

        def body(t, carry):
            for r in range(3):
                c = 3 * t + r
                k = r
                kp = (k + 1) % 3
                _swait(k)
                _cpidx(c, k)
                _gather(c, k)
                _gwait(kp)
                _scat(kp)
            return carry

        lax.fori_loop(1, (_MCPS - 3) // 3 + 1, body, 0)
        _swait(0)
        _cpidx(_MCPS - 2, 0)
        _gather(_MCPS - 2, 0)
        _gwait(1)
        _scat(1)
        _swait(1)
        _cpidx(_MCPS - 1, 1)
        _gather(_MCPS - 1, 1)
        _gwait(2)
        _scat(2)
        _gwait(0)
        _scat(0)
        _gwait(1)
        _scat(1)
        _swait(2)
        _swait(0)
        _swait(1)

    plsc.subcore_barrier()
    pltpu.sync_copy(acc_sh.at[pl.ds(sid * _STRIPE, _STRIPE)],
                    out_hbm.at[cid, pl.ds(sid * _STRIPE, _STRIPE)])

    @pl.when(sid == _NS - 1)
    def _():
        pltpu.sync_copy(acc_sh.at[pl.ds(_TAIL_OFF, _TAIL)],
                        out_hbm.at[cid, pl.ds(_TAIL_OFF, _TAIL)])


_mp = pl.kernel(
    _mp_body,
    out_type=jax.ShapeDtypeStruct((_NC, _N, _H), jnp.float32),
    mesh=_MESH,
    scratch_types=[
        pltpu.VMEM_SHARED((_N, _H), jnp.float32),
        pltpu.VMEM((_MST,), jnp.int32),
        pltpu.VMEM((_MST,), jnp.int32),
        pltpu.VMEM((_CH,), jnp.int32),
        pltpu.VMEM((_CH,), jnp.int32),
        pltpu.VMEM((_CH,), jnp.int32),
        pltpu.VMEM((_CH, _H), jnp.float32),
        pltpu.VMEM((_CH, _H), jnp.float32),
        pltpu.VMEM((_CH, _H), jnp.float32),
        pltpu.SemaphoreType.DMA,
        pltpu.SemaphoreType.DMA,
        pltpu.SemaphoreType.DMA,
        pltpu.SemaphoreType.DMA,
        pltpu.SemaphoreType.DMA,
        pltpu.SemaphoreType.DMA,
    ],
)


def _emb_body(nt_ref, feats_ref, degp_ref, w1_ref, b1_ref, w2_ref, b2_ref,
              wg0_ref, u0_ref, dinv_ref):
    oh = (nt_ref[:] == lax.broadcasted_iota(jnp.int32, (1, _NT), 1))
    x_t = jnp.dot(oh.astype(jnp.float32), w1_ref[:],
                  preferred_element_type=jnp.float32) + b1_ref[:]
    x_num = jnp.dot(feats_ref[:], w2_ref[:],
                    preferred_element_type=jnp.float32) + b2_ref[:]
    deg = degp_ref[0, :, 0:1] + degp_ref[1, :, 0:1] - 1.0
    dinv = lax.rsqrt(deg)
    z = jnp.concatenate([x_t, x_num], axis=1)
    h0 = jnp.dot(z, wg0_ref[:], preferred_element_type=jnp.float32)
    u0_ref[:] = h0 * dinv
    dinv_ref[:] = dinv


_emb = pl.pallas_call(
    _emb_body,
    out_shape=[
        jax.ShapeDtypeStruct((_N, _H), jnp.float32),
        jax.ShapeDtypeStruct((_N, 1), jnp.float32),
    ],
)


def _comb_body(accp_ref, u_ref, dinv_ref, b_ref, gam_ref, bet_ref, w_ref,
               out_ref):
    s = accp_ref[0] + accp_ref[1] - u_ref[:]
    t = s * dinv_ref[:] + b_ref[:]
    mu = jnp.mean(t, axis=0, keepdims=True)
    var = jnp.mean((t - mu) * (t - mu), axis=0, keepdims=True)
    zn = (t - mu) * lax.rsqrt(var + 1e-5) * gam_ref[:] + bet_ref[:]
    z = jnp.maximum(zn, 0.0)
    out_ref[:] = jnp.dot(z, w_ref[:],
                         preferred_element_type=jnp.float32) * dinv_ref[:]


_comb = pl.pallas_call(
    _comb_body,
    out_shape=jax.ShapeDtypeStruct((_N, _H), jnp.float32),
)


def _final_body(accp_ref, u_ref, dinv_ref, b_ref, batch_ref, wo_ref, bo_ref,
                out_ref):
    s = accp_ref[0] + accp_ref[1] - u_ref[:]
    t = s * dinv_ref[:] + b_ref[:]
    z = jnp.maximum(t, 0.0)
    ohb = (batch_ref[:] == lax.broadcasted_iota(jnp.int32, (1, _G), 1))
    zz = jnp.concatenate([z, jnp.ones((_N, 1), jnp.float32)], axis=1)
    s2 = lax.dot_general(ohb.astype(jnp.float32), zz,
                         (((0,), (0,)), ((), ())),
                         preferred_element_type=jnp.float32)
    pooled = s2[:, :_H] / jnp.maximum(s2[:, _H:_H + 1], 1.0)
    out_ref[:] = jnp.dot(pooled, wo_ref[:],
                         preferred_element_type=jnp.float32) + bo_ref[:]


_final = pl.pallas_call(
    _final_body,
    out_shape=jax.ShapeDtypeStruct((_G, 4), jnp.float32),
)


def kernel(node_type, c, gm, pos, r, vid, edge_index, batch, W1, b1, W2, b2,
           Wg0, bg0, Wg1, bg1, Wg2, bg2, gam0, bet0, gam1, bet1, Wo, bo):
    nt2 = node_type.reshape(_N, 1).astype(jnp.int32)
    feats = jnp.stack([c, gm, pos, r, vid], axis=-1)
    srcr = edge_index[0].astype(jnp.int32).reshape(_E)
    dstr = edge_index[1].astype(jnp.int32).reshape(_E)
    batch2 = batch.reshape(_N, 1).astype(jnp.int32)
    ones16 = jnp.ones((_N, 16), jnp.float32)

    degp = _deg(dstr, ones16)
    u0, dinv = _emb(nt2, feats, degp, W1, b1.reshape(1, _H), W2,
                    b2.reshape(1, _H), Wg0)
    acc0 = _mp(u0, srcr, dstr)
    u1 = _comb(acc0, u0, dinv, bg0.reshape(1, _H), gam0.reshape(1, _H),
               bet0.reshape(1, _H), Wg1)
    acc1 = _mp(u1, srcr, dstr)
    u2 = _comb(acc1, u1, dinv, bg1.reshape(1, _H), gam1.reshape(1, _H),
               bet1.reshape(1, _H), Wg2)
    acc2 = _mp(u2, srcr, dstr)
    return _final(acc2, u2, dinv, bg2.reshape(1, _H), batch2, Wo,
                  bo.reshape(1, 4))


# speedup vs baseline: 34.9372x; 1.0254x over previous
import functools

import jax
import jax.numpy as jnp
from jax import lax
from jax.experimental import pallas as pl
from jax.experimental.pallas import tpu as pltpu
from jax.experimental.pallas import tpu_sc as plsc

_N = 10000
_E = 640000
_H = 128
_NT = 16
_G = 64

_NC = 2
_NS = 16
_NW = _NC * _NS
_CH = 80
_EPT = _E // _NW
_RPT = _EPT // _CH
_HALF = _EPT // 2
_CPS = _HALF // _CH
_MST = 4000
_MCPS = _MST // _CH
_MNST = _EPT // _MST
_STRIPE = 624
_TAIL = _N - _NS * _STRIPE
_TAIL_OFF = _NS * _STRIPE

_MESH = plsc.VectorSubcoreMesh(core_axis_name="c", subcore_axis_name="s")


def _deg_body(dstr_hbm, ones_hbm, out_hbm, acc_sh, didx_st, db0, db1, db2,
              ones_v, ss0, ss1, ss2):
    cid = lax.axis_index("c")
    sid = lax.axis_index("s")
    wid = sid * _NC + cid
    pltpu.sync_copy(ones_hbm.at[pl.ds(sid * _STRIPE, _STRIPE)],
                    acc_sh.at[pl.ds(sid * _STRIPE, _STRIPE)])

    @pl.when(sid == _NS - 1)
    def _():
        pltpu.sync_copy(ones_hbm.at[pl.ds(_TAIL_OFF, _TAIL)],
                        acc_sh.at[pl.ds(_TAIL_OFF, _TAIL)])

    pltpu.sync_copy(ones_hbm.at[pl.ds(0, _CH)], ones_v)
    plsc.subcore_barrier()

    dbs = (db0, db1, db2)
    sss = (ss0, ss1, ss2)

    def _cpidx(c, k):
        for q in range(_CH // 16):
            dbs[k][pl.ds(16 * q, 16)] = didx_st[pl.ds(c * _CH + 16 * q, 16)]

    def _scat(k):
        pltpu.async_copy(ones_v, acc_sh.at[dbs[k]], sss[k], add=True)

    def _swait(k):
        pltpu.make_async_copy(ones_v, acc_sh.at[dbs[k]], sss[k]).wait()

    for s in range(2):
        sbase = wid * _EPT + s * _HALF
        pltpu.sync_copy(dstr_hbm.at[pl.ds(sbase, _HALF)], didx_st)
        for c in range(3):
            _cpidx(c, c)
            _scat(c)

        def body(t, carry):
            for r in range(3):
                c = 3 * t + r
                _swait(r)
                _cpidx(c, r)
                _scat(r)
            return carry

        lax.fori_loop(1, (_CPS - 3) // 3 + 1, body, 0)
        for c, k in ((_CPS - 2, 0), (_CPS - 1, 1)):
            _swait(k)
            _cpidx(c, k)
            _scat(k)
        _swait(2)
        _swait(0)
        _swait(1)

    plsc.subcore_barrier()
    pltpu.sync_copy(acc_sh.at[pl.ds(sid * _STRIPE, _STRIPE)],
                    out_hbm.at[cid, pl.ds(sid * _STRIPE, _STRIPE)])

    @pl.when(sid == _NS - 1)
    def _():
        pltpu.sync_copy(acc_sh.at[pl.ds(_TAIL_OFF, _TAIL)],
                        out_hbm.at[cid, pl.ds(_TAIL_OFF, _TAIL)])


_deg = pl.kernel(
    _deg_body,
    out_type=jax.ShapeDtypeStruct((_NC, _N, 16), jnp.float32),
    mesh=_MESH,
    scratch_types=[
        pltpu.VMEM_SHARED((_N, 16), jnp.float32),
        pltpu.VMEM((_HALF,), jnp.int32),
        pltpu.VMEM((_CH,), jnp.int32),
        pltpu.VMEM((_CH,), jnp.int32),
        pltpu.VMEM((_CH,), jnp.int32),
        pltpu.VMEM((_CH, 16), jnp.float32),
        pltpu.SemaphoreType.DMA,
        pltpu.SemaphoreType.DMA,
        pltpu.SemaphoreType.DMA,
    ],
)


def _mp_body(u_hbm, srcr_hbm, dstr_hbm, out_hbm, acc_sh, sidx_st, didx_st,
             db0, db1, db2, rows0, rows1, rows2, g0, g1, g2, ss0, ss1, ss2):
    cid = lax.axis_index("c")
    sid = lax.axis_index("s")
    wid = sid * _NC + cid
    pltpu.sync_copy(u_hbm.at[pl.ds(sid * _STRIPE, _STRIPE)],
                    acc_sh.at[pl.ds(sid * _STRIPE, _STRIPE)])

    @pl.when(sid == _NS - 1)
    def _():
        pltpu.sync_copy(u_hbm.at[pl.ds(_TAIL_OFF, _TAIL)],
                        acc_sh.at[pl.ds(_TAIL_OFF, _TAIL)])

    plsc.subcore_barrier()

    dbs = (db0, db1, db2)
    rows = (rows0, rows1, rows2)
    gs = (g0, g1, g2)
    sss = (ss0, ss1, ss2)

    def _gather(c, k):
        pltpu.async_copy(u_hbm.at[sidx_st.at[pl.ds(c * _CH, _CH)]],
                         rows[k], gs[k])

    def _gwait(k):
        pltpu.make_async_copy(u_hbm.at[sidx_st.at[pl.ds(0, _CH)]],
                              rows[k], gs[k]).wait()

    def _cpidx(c, k):
        for q in range(_CH // 16):
            dbs[k][pl.ds(16 * q, 16)] = didx_st[pl.ds(c * _CH + 16 * q, 16)]

    def _scat(k):
        pltpu.async_copy(rows[k], acc_sh.at[dbs[k]], sss[k], add=True)

    def _swait(k):
        pltpu.make_async_copy(rows[k], acc_sh.at[dbs[k]], sss[k]).wait()

    for s in range(_MNST):
        sbase = wid * _EPT + s * _MST
        pltpu.sync_copy(srcr_hbm.at[pl.ds(sbase, _MST)], sidx_st)
        pltpu.sync_copy(dstr_hbm.at[pl.ds(sbase, _MST)], didx_st)
        _cpidx(0, 0)
        _gather(0, 0)
        _cpidx(1, 1)
        _gather(1, 1)
        _cpidx(2, 2)
        _gather(2, 2)
        _gwait(0)
        _scat(0)

        def body(t, carry):
            for r in range(3):
                c = 3 * t + r
                k = r
                kp = (k + 1) % 3
                _swait(k)
                _cpidx(c, k)
                _gather(c, k)
                _gwait(kp)
                _scat(kp)
            return carry

        lax.fori_loop(1, (_MCPS - 3) // 3 + 1, body, 0)
        _swait(0)
        _cpidx(_MCPS - 2, 0)
        _gather(_MCPS - 2, 0)
        _gwait(1)
        _scat(1)
        _swait(1)
        _cpidx(_MCPS - 1, 1)
        _gather(_MCPS - 1, 1)
        _gwait(2)
        _scat(2)
        _gwait(0)
        _scat(0)
        _gwait(1)
        _scat(1)
        _swait(2)
        _swait(0)
        _swait(1)

    plsc.subcore_barrier()
    pltpu.sync_copy(acc_sh.at[pl.ds(sid * _STRIPE, _STRIPE)],
                    out_hbm.at[cid, pl.ds(sid * _STRIPE, _STRIPE)])

    @pl.when(sid == _NS - 1)
    def _():
        pltpu.sync_copy(acc_sh.at[pl.ds(_TAIL_OFF, _TAIL)],
                        out_hbm.at[cid, pl.ds(_TAIL_OFF, _TAIL)])


_mp = pl.kernel(
    _mp_body,
    out_type=jax.ShapeDtypeStruct((_NC, _N, _H), jnp.float32),
    mesh=_MESH,
    scratch_types=[
        pltpu.VMEM_SHARED((_N, _H), jnp.float32),
        pltpu.VMEM((_MST,), jnp.int32),
        pltpu.VMEM((_MST,), jnp.int32),
        pltpu.VMEM((_CH,), jnp.int32),
        pltpu.VMEM((_CH,), jnp.int32),
        pltpu.VMEM((_CH,), jnp.int32),
        pltpu.VMEM((_CH, _H), jnp.float32),
        pltpu.VMEM((_CH, _H), jnp.float32),
        pltpu.VMEM((_CH, _H), jnp.float32),
        pltpu.SemaphoreType.DMA,
        pltpu.SemaphoreType.DMA,
        pltpu.SemaphoreType.DMA,
        pltpu.SemaphoreType.DMA,
        pltpu.SemaphoreType.DMA,
        pltpu.SemaphoreType.DMA,
    ],
)


def _emb_body(nt_ref, feats_ref, degp_ref, w1a_ref, w2b_ref, bc_ref,
              u0_ref, dinv_ref):
    oh = (nt_ref[:] == lax.broadcasted_iota(jnp.int32, (1, _NT), 1))
    h0 = (jnp.dot(oh.astype(jnp.float32), w1a_ref[:],
                  preferred_element_type=jnp.float32)
          + jnp.dot(feats_ref[:], w2b_ref[:],
                    preferred_element_type=jnp.float32) + bc_ref[:])
    deg = degp_ref[0, :, 0:1] + degp_ref[1, :, 0:1] - 1.0
    dinv = lax.rsqrt(deg)
    u0_ref[:] = h0 * dinv
    dinv_ref[:] = dinv


_emb = pl.pallas_call(
    _emb_body,
    out_shape=[
        jax.ShapeDtypeStruct((_N, _H), jnp.float32),
        jax.ShapeDtypeStruct((_N, 1), jnp.float32),
    ],
)


def _comb_body(accp_ref, u_ref, dinv_ref, b_ref, gam_ref, bet_ref, w_ref,
               out_ref):
    s = accp_ref[0] + accp_ref[1] - u_ref[:]
    t = s * dinv_ref[:] + b_ref[:]
    mu = jnp.mean(t, axis=0, keepdims=True)
    var = jnp.mean((t - mu) * (t - mu), axis=0, keepdims=True)
    zn = (t - mu) * lax.rsqrt(var + 1e-5) * gam_ref[:] + bet_ref[:]
    z = jnp.maximum(zn, 0.0)
    out_ref[:] = jnp.dot(z, w_ref[:],
                         preferred_element_type=jnp.float32) * dinv_ref[:]


_comb = pl.pallas_call(
    _comb_body,
    out_shape=jax.ShapeDtypeStruct((_N, _H), jnp.float32),
)


def _final_body(accp_ref, u_ref, dinv_ref, b_ref, batch_ref, wo_ref, bo_ref,
                out_ref):
    s = accp_ref[0] + accp_ref[1] - u_ref[:]
    t = s * dinv_ref[:] + b_ref[:]
    z = jnp.maximum(t, 0.0)
    ohb = (batch_ref[:] == lax.broadcasted_iota(jnp.int32, (1, _G), 1))
    zz = jnp.concatenate([z, jnp.ones((_N, 1), jnp.float32)], axis=1)
    s2 = lax.dot_general(ohb.astype(jnp.float32), zz,
                         (((0,), (0,)), ((), ())),
                         preferred_element_type=jnp.float32)
    pooled = s2[:, :_H] / jnp.maximum(s2[:, _H:_H + 1], 1.0)
    out_ref[:] = jnp.dot(pooled, wo_ref[:],
                         preferred_element_type=jnp.float32) + bo_ref[:]


_final = pl.pallas_call(
    _final_body,
    out_shape=jax.ShapeDtypeStruct((_G, 4), jnp.float32),
)


def kernel(node_type, c, gm, pos, r, vid, edge_index, batch, W1, b1, W2, b2,
           Wg0, bg0, Wg1, bg1, Wg2, bg2, gam0, bet0, gam1, bet1, Wo, bo):
    nt2 = node_type.reshape(_N, 1).astype(jnp.int32)
    feats = jnp.stack([c, gm, pos, r, vid], axis=-1)
    srcr = edge_index[0].astype(jnp.int32).reshape(_E)
    dstr = edge_index[1].astype(jnp.int32).reshape(_E)
    batch2 = batch.reshape(_N, 1).astype(jnp.int32)
    ones16 = jnp.ones((_N, 16), jnp.float32)

    degp = _deg(dstr, ones16)
    a0, a1 = Wg0[:_H], Wg0[_H:]
    w1a = W1 @ a0
    w2b = W2 @ a1
    bc = (b1 @ a0 + b2 @ a1).reshape(1, _H)
    u0, dinv = _emb(nt2, feats, degp, w1a, w2b, bc)
    acc0 = _mp(u0, srcr, dstr)
    u1 = _comb(acc0, u0, dinv, bg0.reshape(1, _H), gam0.reshape(1, _H),
               bet0.reshape(1, _H), Wg1)
    acc1 = _mp(u1, srcr, dstr)
    u2 = _comb(acc1, u1, dinv, bg1.reshape(1, _H), gam1.reshape(1, _H),
               bet1.reshape(1, _H), Wg2)
    acc2 = _mp(u2, srcr, dstr)
    return _final(acc2, u2, dinv, bg2.reshape(1, _H), batch2, Wo,
                  bo.reshape(1, 4))


# speedup vs baseline: 35.7242x; 1.0225x over previous
import functools

import jax
import jax.numpy as jnp
from jax import lax
from jax.experimental import pallas as pl
from jax.experimental.pallas import tpu as pltpu
from jax.experimental.pallas import tpu_sc as plsc

_N = 10000
_E = 640000
_H = 128
_NT = 16
_G = 64

_NC = 2
_NS = 16
_NW = _NC * _NS
_CH = 80
_EPT = _E // _NW
_RPT = _EPT // _CH
_HALF = _EPT // 2
_CPS = _HALF // _CH
_MST = 4000
_MCPS = _MST // _CH
_MNST = _EPT // _MST
_STRIPE = 624
_TAIL = _N - _NS * _STRIPE
_TAIL_OFF = _NS * _STRIPE

_MESH = plsc.VectorSubcoreMesh(core_axis_name="c", subcore_axis_name="s")


def _deg_body(dstr_hbm, ones_hbm, out_hbm, acc_sh, didx_a, didx_b,
              db0, db1, db2, ones_v, ss0, ss1, ss2, pf0, pf1):
    cid = lax.axis_index("c")
    sid = lax.axis_index("s")
    wid = sid * _NC + cid
    pltpu.sync_copy(ones_hbm.at[pl.ds(sid * _STRIPE, _STRIPE)],
                    acc_sh.at[pl.ds(sid * _STRIPE, _STRIPE)])

    @pl.when(sid == _NS - 1)
    def _():
        pltpu.sync_copy(ones_hbm.at[pl.ds(_TAIL_OFF, _TAIL)],
                        acc_sh.at[pl.ds(_TAIL_OFF, _TAIL)])

    pltpu.sync_copy(ones_hbm.at[pl.ds(0, _CH)], ones_v)
    plsc.subcore_barrier()

    dbs = (db0, db1, db2)
    sss = (ss0, ss1, ss2)
    didxs = (didx_a, didx_b)
    pfs = (pf0, pf1)

    def _scat(k):
        pltpu.async_copy(ones_v, acc_sh.at[dbs[k]], sss[k], add=True)

    def _swait(k):
        pltpu.make_async_copy(ones_v, acc_sh.at[dbs[k]], sss[k]).wait()

    def _pf_issue(s):
        sbase = wid * _EPT + s * _HALF
        pltpu.async_copy(dstr_hbm.at[pl.ds(sbase, _HALF)], didxs[s % 2],
                         pfs[s % 2])

    def _pf_wait(pb):
        pltpu.make_async_copy(dstr_hbm.at[pl.ds(0, _HALF)], didxs[pb],
                              pfs[pb]).wait()

    _pf_issue(0)
    for s in range(2):
        didx_st = didxs[s % 2]
        if s == 0:
            _pf_issue(1)
        _pf_wait(s % 2)

        def _cpidx(c, k):
            for q in range(_CH // 16):
                dbs[k][pl.ds(16 * q, 16)] = \
                    didx_st[pl.ds(c * _CH + 16 * q, 16)]

        for c in range(3):
            _cpidx(c, c)
            _scat(c)

        def body(t, carry):
            for r in range(3):
                c = 3 * t + r
                _swait(r)
                _cpidx(c, r)
                _scat(r)
            return carry

        lax.fori_loop(1, (_CPS - 3) // 3 + 1, body, 0)
        for c, k in ((_CPS - 2, 0), (_CPS - 1, 1)):
            _swait(k)
            _cpidx(c, k)
            _scat(k)
        _swait(2)
        _swait(0)
        _swait(1)

    plsc.subcore_barrier()
    pltpu.sync_copy(acc_sh.at[pl.ds(sid * _STRIPE, _STRIPE)],
                    out_hbm.at[cid, pl.ds(sid * _STRIPE, _STRIPE)])

    @pl.when(sid == _NS - 1)
    def _():
        pltpu.sync_copy(acc_sh.at[pl.ds(_TAIL_OFF, _TAIL)],
                        out_hbm.at[cid, pl.ds(_TAIL_OFF, _TAIL)])


_deg = pl.kernel(
    _deg_body,
    out_type=jax.ShapeDtypeStruct((_NC, _N, 16), jnp.float32),
    mesh=_MESH,
    scratch_types=[
        pltpu.VMEM_SHARED((_N, 16), jnp.float32),
        pltpu.VMEM((_HALF,), jnp.int32),
        pltpu.VMEM((_HALF,), jnp.int32),
        pltpu.VMEM((_CH,), jnp.int32),
        pltpu.VMEM((_CH,), jnp.int32),
        pltpu.VMEM((_CH,), jnp.int32),
        pltpu.VMEM((_CH, 16), jnp.float32),
        pltpu.SemaphoreType.DMA,
        pltpu.SemaphoreType.DMA,
        pltpu.SemaphoreType.DMA,
        pltpu.SemaphoreType.DMA,
        pltpu.SemaphoreType.DMA,
    ],
)


def _mp_body(u_hbm, srcr_hbm, dstr_hbm, out_hbm, acc_sh, sidx_a, sidx_b,
             didx_a, didx_b, db0, db1, db2, rows0, rows1, rows2,
             g0, g1, g2, ss0, ss1, ss2, pf0, pf1):
    cid = lax.axis_index("c")
    sid = lax.axis_index("s")
    wid = sid * _NC + cid
    pltpu.sync_copy(u_hbm.at[pl.ds(sid * _STRIPE, _STRIPE)],
                    acc_sh.at[pl.ds(sid * _STRIPE, _STRIPE)])

    @pl.when(sid == _NS - 1)
    def _():
        pltpu.sync_copy(u_hbm.at[pl.ds(_TAIL_OFF, _TAIL)],
                        acc_sh.at[pl.ds(_TAIL_OFF, _TAIL)])

    plsc.subcore_barrier()

    dbs = (db0, db1, db2)
    rows = (rows0, rows1, rows2)
    gs = (g0, g1, g2)
    sss = (ss0, ss1, ss2)
    sidxs = (sidx_a, sidx_b)
    didxs = (didx_a, didx_b)
    pfs = (pf0, pf1)

    def _pf_issue(s):
        sbase = wid * _EPT + s * _MST
        pb = s % 2
        pltpu.async_copy(srcr_hbm.at[pl.ds(sbase, _MST)], sidxs[pb], pfs[pb])
        pltpu.async_copy(dstr_hbm.at[pl.ds(sbase, _MST)], didxs[pb], pfs[pb])

    def _pf_wait(pb):
        sbase = wid * _EPT
        pltpu.make_async_copy(srcr_hbm.at[pl.ds(sbase, _MST)],
                              sidxs[pb], pfs[pb]).wait()
        pltpu.make_async_copy(dstr_hbm.at[pl.ds(sbase, _MST)],
                              didxs[pb], pfs[pb]).wait()

    _pf_issue(0)
    for s in range(_MNST):
        pb = s % 2
        sidx_st = sidxs[pb]
        didx_st = didxs[pb]
        if s + 1 < _MNST:
            _pf_issue(s + 1)
        _pf_wait(pb)

        def _gather(c, k):
            pltpu.async_copy(u_hbm.at[sidx_st.at[pl.ds(c * _CH, _CH)]],
                             rows[k], gs[k])

        def _gwait(k):
            pltpu.make_async_copy(u_hbm.at[sidx_st.at[pl.ds(0, _CH)]],
                                  rows[k], gs[k]).wait()

        def _cpidx(c, k):
            for q in range(_CH // 16):
                dbs[k][pl.ds(16 * q, 16)] = \
                    didx_st[pl.ds(c * _CH + 16 * q, 16)]

        def _scat(k):
            pltpu.async_copy(rows[k], acc_sh.at[dbs[k]], sss[k], add=True)

        def _swait(k):
            pltpu.make_async_copy(rows[k], acc_sh.at[dbs[k]], sss[k]).wait()

        _cpidx(0, 0)
        _gather(0, 0)
        _cpidx(1, 1)
        _gather(1, 1)
        _cpidx(2, 2)
        _gather(2, 2)
        _gwait(0)
        _scat(0)

        def body(t, carry):
            for r in range(3):
                c = 3 * t + r
                k = r
                kp = (k + 1) % 3
                _swait(k)
                _cpidx(c, k)
                _gather(c, k)
                _gwait(kp)
                _scat(kp)
            return carry

        lax.fori_loop(1, (_MCPS - 3) // 3 + 1, body, 0)
        _swait(0)
        _cpidx(_MCPS - 2, 0)
        _gather(_MCPS - 2, 0)
        _gwait(1)
        _scat(1)
        _swait(1)
        _cpidx(_MCPS - 1, 1)
        _gather(_MCPS - 1, 1)
        _gwait(2)
        _scat(2)
        _gwait(0)
        _scat(0)
        _gwait(1)
        _scat(1)
        _swait(2)
        _swait(0)
        _swait(1)

    plsc.subcore_barrier()
    pltpu.sync_copy(acc_sh.at[pl.ds(sid * _STRIPE, _STRIPE)],
                    out_hbm.at[cid, pl.ds(sid * _STRIPE, _STRIPE)])

    @pl.when(sid == _NS - 1)
    def _():
        pltpu.sync_copy(acc_sh.at[pl.ds(_TAIL_OFF, _TAIL)],
                        out_hbm.at[cid, pl.ds(_TAIL_OFF, _TAIL)])


_mp = pl.kernel(
    _mp_body,
    out_type=jax.ShapeDtypeStruct((_NC, _N, _H), jnp.float32),
    mesh=_MESH,
    scratch_types=[
        pltpu.VMEM_SHARED((_N, _H), jnp.float32),
        pltpu.VMEM((_MST,), jnp.int32),
        pltpu.VMEM((_MST,), jnp.int32),
        pltpu.VMEM((_MST,), jnp.int32),
        pltpu.VMEM((_MST,), jnp.int32),
        pltpu.VMEM((_CH,), jnp.int32),
        pltpu.VMEM((_CH,), jnp.int32),
        pltpu.VMEM((_CH,), jnp.int32),
        pltpu.VMEM((_CH, _H), jnp.float32),
        pltpu.VMEM((_CH, _H), jnp.float32),
        pltpu.VMEM((_CH, _H), jnp.float32),
        pltpu.SemaphoreType.DMA,
        pltpu.SemaphoreType.DMA,
        pltpu.SemaphoreType.DMA,
        pltpu.SemaphoreType.DMA,
        pltpu.SemaphoreType.DMA,
        pltpu.SemaphoreType.DMA,
        pltpu.SemaphoreType.DMA,
        pltpu.SemaphoreType.DMA,
    ],
)


def _emb_body(nt_ref, feats_ref, degp_ref, w1a_ref, w2b_ref, bc_ref,
              u0_ref, dinv_ref):
    oh = (nt_ref[:] == lax.broadcasted_iota(jnp.int32, (1, _NT), 1))
    h0 = (jnp.dot(oh.astype(jnp.float32), w1a_ref[:],
                  preferred_element_type=jnp.float32)
          + jnp.dot(feats_ref[:], w2b_ref[:],
                    preferred_element_type=jnp.float32) + bc_ref[:])
    deg = degp_ref[0, :, 0:1] + degp_ref[1, :, 0:1] - 1.0
    dinv = lax.rsqrt(deg)
    u0_ref[:] = h0 * dinv
    dinv_ref[:] = dinv


_emb = pl.pallas_call(
    _emb_body,
    out_shape=[
        jax.ShapeDtypeStruct((_N, _H), jnp.float32),
        jax.ShapeDtypeStruct((_N, 1), jnp.float32),
    ],
)


def _comb_body(accp_ref, u_ref, dinv_ref, b_ref, gam_ref, bet_ref, w_ref,
               out_ref):
    s = accp_ref[0] + accp_ref[1] - u_ref[:]
    t = s * dinv_ref[:] + b_ref[:]
    mu = jnp.mean(t, axis=0, keepdims=True)
    var = jnp.mean((t - mu) * (t - mu), axis=0, keepdims=True)
    zn = (t - mu) * lax.rsqrt(var + 1e-5) * gam_ref[:] + bet_ref[:]
    z = jnp.maximum(zn, 0.0)
    out_ref[:] = jnp.dot(z, w_ref[:],
                         preferred_element_type=jnp.float32) * dinv_ref[:]


_comb = pl.pallas_call(
    _comb_body,
    out_shape=jax.ShapeDtypeStruct((_N, _H), jnp.float32),
)


def _final_body(accp_ref, u_ref, dinv_ref, b_ref, batch_ref, wo_ref, bo_ref,
                out_ref):
    s = accp_ref[0] + accp_ref[1] - u_ref[:]
    t = s * dinv_ref[:] + b_ref[:]
    z = jnp.maximum(t, 0.0)
    ohb = (batch_ref[:] == lax.broadcasted_iota(jnp.int32, (1, _G), 1))
    zz = jnp.concatenate([z, jnp.ones((_N, 1), jnp.float32)], axis=1)
    s2 = lax.dot_general(ohb.astype(jnp.float32), zz,
                         (((0,), (0,)), ((), ())),
                         preferred_element_type=jnp.float32)
    pooled = s2[:, :_H] / jnp.maximum(s2[:, _H:_H + 1], 1.0)
    out_ref[:] = jnp.dot(pooled, wo_ref[:],
                         preferred_element_type=jnp.float32) + bo_ref[:]


_final = pl.pallas_call(
    _final_body,
    out_shape=jax.ShapeDtypeStruct((_G, 4), jnp.float32),
)


def kernel(node_type, c, gm, pos, r, vid, edge_index, batch, W1, b1, W2, b2,
           Wg0, bg0, Wg1, bg1, Wg2, bg2, gam0, bet0, gam1, bet1, Wo, bo):
    nt2 = node_type.reshape(_N, 1).astype(jnp.int32)
    feats = jnp.stack([c, gm, pos, r, vid], axis=-1)
    srcr = edge_index[0].astype(jnp.int32).reshape(_E)
    dstr = edge_index[1].astype(jnp.int32).reshape(_E)
    batch2 = batch.reshape(_N, 1).astype(jnp.int32)
    ones16 = jnp.ones((_N, 16), jnp.float32)

    degp = _deg(dstr, ones16)
    a0, a1 = Wg0[:_H], Wg0[_H:]
    w1a = W1 @ a0
    w2b = W2 @ a1
    bc = (b1 @ a0 + b2 @ a1).reshape(1, _H)
    u0, dinv = _emb(nt2, feats, degp, w1a, w2b, bc)
    acc0 = _mp(u0, srcr, dstr)
    u1 = _comb(acc0, u0, dinv, bg0.reshape(1, _H), gam0.reshape(1, _H),
               bet0.reshape(1, _H), Wg1)
    acc1 = _mp(u1, srcr, dstr)
    u2 = _comb(acc1, u1, dinv, bg1.reshape(1, _H), gam1.reshape(1, _H),
               bet1.reshape(1, _H), Wg2)
    acc2 = _mp(u2, srcr, dstr)
    return _final(acc2, u2, dinv, bg2.reshape(1, _H), batch2, Wo,
                  bo.reshape(1, 4))


# speedup vs baseline: 35.8301x; 1.0030x over previous
import jax
import jax.numpy as jnp
from jax import lax
from jax.experimental import pallas as pl
from jax.experimental.pallas import tpu as pltpu
from jax.experimental.pallas import tpu_sc as plsc

_N = 10000
_E = 640000
_H = 128
_NT = 16
_G = 64

_NC = 2
_NS = 16
_NW = _NC * _NS
_CH = 80
_EPT = _E // _NW
_HALF = _EPT // 2
_CPS = _HALF // _CH
_MST = 4000
_MCPS = _MST // _CH
_MNST = _EPT // _MST
_STRIPE = 624
_TAIL = _N - _NS * _STRIPE
_TAIL_OFF = _NS * _STRIPE

_MESH = plsc.VectorSubcoreMesh(core_axis_name="c", subcore_axis_name="s")


def _deg_body(dstr_hbm, ones_hbm, out_hbm, acc_sh, didx_a, didx_b,
              db0, db1, db2, ones_v, ss0, ss1, ss2, pf0, pf1):
    cid = lax.axis_index("c")
    sid = lax.axis_index("s")
    wid = sid * _NC + cid
    pltpu.sync_copy(ones_hbm.at[pl.ds(sid * _STRIPE, _STRIPE)],
                    acc_sh.at[pl.ds(sid * _STRIPE, _STRIPE)])

    @pl.when(sid == _NS - 1)
    def _():
        pltpu.sync_copy(ones_hbm.at[pl.ds(_TAIL_OFF, _TAIL)],
                        acc_sh.at[pl.ds(_TAIL_OFF, _TAIL)])

    pltpu.sync_copy(ones_hbm.at[pl.ds(0, _CH)], ones_v)
    plsc.subcore_barrier()

    dbs = (db0, db1, db2)
    sss = (ss0, ss1, ss2)
    didxs = (didx_a, didx_b)
    pfs = (pf0, pf1)

    def _scat(k):
        pltpu.async_copy(ones_v, acc_sh.at[dbs[k]], sss[k], add=True)

    def _swait(k):
        pltpu.make_async_copy(ones_v, acc_sh.at[dbs[k]], sss[k]).wait()

    def _pf_issue(s):
        sbase = wid * _EPT + s * _HALF
        pltpu.async_copy(dstr_hbm.at[pl.ds(sbase, _HALF)], didxs[s % 2],
                         pfs[s % 2])

    def _pf_wait(pb):
        pltpu.make_async_copy(dstr_hbm.at[pl.ds(0, _HALF)], didxs[pb],
                              pfs[pb]).wait()

    _pf_issue(0)
    for s in range(2):
        didx_st = didxs[s % 2]
        if s == 0:
            _pf_issue(1)
        _pf_wait(s % 2)

        def _cpidx(c, k):
            for q in range(_CH // 16):
                dbs[k][pl.ds(16 * q, 16)] = \
                    didx_st[pl.ds(c * _CH + 16 * q, 16)]

        for c in range(3):
            _cpidx(c, c)
            _scat(c)

        def body(t, carry):
            for r in range(3):
                c = 3 * t + r
                _swait(r)
                _cpidx(c, r)
                _scat(r)
            return carry

        lax.fori_loop(1, (_CPS - 3) // 3 + 1, body, 0)
        for c, k in ((_CPS - 2, 0), (_CPS - 1, 1)):
            _swait(k)
            _cpidx(c, k)
            _scat(k)
        _swait(2)
        _swait(0)
        _swait(1)

    plsc.subcore_barrier()
    pltpu.sync_copy(acc_sh.at[pl.ds(sid * _STRIPE, _STRIPE)],
                    out_hbm.at[cid, pl.ds(sid * _STRIPE, _STRIPE)])

    @pl.when(sid == _NS - 1)
    def _():
        pltpu.sync_copy(acc_sh.at[pl.ds(_TAIL_OFF, _TAIL)],
                        out_hbm.at[cid, pl.ds(_TAIL_OFF, _TAIL)])


_deg = pl.kernel(
    _deg_body,
    out_type=jax.ShapeDtypeStruct((_NC, _N, 16), jnp.float32),
    mesh=_MESH,
    scratch_types=[
        pltpu.VMEM_SHARED((_N, 16), jnp.float32),
        pltpu.VMEM((_HALF,), jnp.int32),
        pltpu.VMEM((_HALF,), jnp.int32),
        pltpu.VMEM((_CH,), jnp.int32),
        pltpu.VMEM((_CH,), jnp.int32),
        pltpu.VMEM((_CH,), jnp.int32),
        pltpu.VMEM((_CH, 16), jnp.float32),
        pltpu.SemaphoreType.DMA,
        pltpu.SemaphoreType.DMA,
        pltpu.SemaphoreType.DMA,
        pltpu.SemaphoreType.DMA,
        pltpu.SemaphoreType.DMA,
    ],
)


def _mp_body(u_hbm, srcr_hbm, dstr_hbm, out_hbm, acc_sh, sidx_a, sidx_b,
             didx_a, didx_b, db0, db1, db2, rows0, rows1, rows2,
             g0, g1, g2, ss0, ss1, ss2, pf0, pf1):
    cid = lax.axis_index("c")
    sid = lax.axis_index("s")
    wid = sid * _NC + cid
    pltpu.sync_copy(u_hbm.at[pl.ds(sid * _STRIPE, _STRIPE)],
                    acc_sh.at[pl.ds(sid * _STRIPE, _STRIPE)])

    @pl.when(sid == _NS - 1)
    def _():
        pltpu.sync_copy(u_hbm.at[pl.ds(_TAIL_OFF, _TAIL)],
                        acc_sh.at[pl.ds(_TAIL_OFF, _TAIL)])

    plsc.subcore_barrier()

    dbs = (db0, db1, db2)
    rows = (rows0, rows1, rows2)
    gs = (g0, g1, g2)
    sss = (ss0, ss1, ss2)
    sidxs = (sidx_a, sidx_b)
    didxs = (didx_a, didx_b)
    pfs = (pf0, pf1)

    def _pf_issue(s):
        sbase = wid * _EPT + s * _MST
        pb = s % 2
        pltpu.async_copy(srcr_hbm.at[pl.ds(sbase, _MST)], sidxs[pb], pfs[pb])
        pltpu.async_copy(dstr_hbm.at[pl.ds(sbase, _MST)], didxs[pb], pfs[pb])

    def _pf_wait(pb):
        sbase = wid * _EPT
        pltpu.make_async_copy(srcr_hbm.at[pl.ds(sbase, _MST)],
                              sidxs[pb], pfs[pb]).wait()
        pltpu.make_async_copy(dstr_hbm.at[pl.ds(sbase, _MST)],
                              didxs[pb], pfs[pb]).wait()

    _pf_issue(0)
    for s in range(_MNST):
        pb = s % 2
        sidx_st = sidxs[pb]
        didx_st = didxs[pb]
        if s + 1 < _MNST:
            _pf_issue(s + 1)
        _pf_wait(pb)

        def _gather(c, k):
            pltpu.async_copy(u_hbm.at[sidx_st.at[pl.ds(c * _CH, _CH)]],
                             rows[k], gs[k])

        def _gwait(k):
            pltpu.make_async_copy(u_hbm.at[sidx_st.at[pl.ds(0, _CH)]],
                                  rows[k], gs[k]).wait()

        def _cpidx(c, k):
            for q in range(_CH // 16):
                dbs[k][pl.ds(16 * q, 16)] = \
                    didx_st[pl.ds(c * _CH + 16 * q, 16)]

        def _scat(k):
            pltpu.async_copy(rows[k], acc_sh.at[dbs[k]], sss[k], add=True)

        def _swait(k):
            pltpu.make_async_copy(rows[k], acc_sh.at[dbs[k]], sss[k]).wait()

        _cpidx(0, 0)
        _gather(0, 0)
        _cpidx(1, 1)
        _gather(1, 1)
        _cpidx(2, 2)
        _gather(2, 2)
        _gwait(0)
        _scat(0)

        def body(t, carry):
            for r in range(3):
                c = 3 * t + r
                k = r
                kp = (k + 1) % 3
                _swait(k)
                _cpidx(c, k)
                _gather(c, k)
                _gwait(kp)
                _scat(kp)
            return carry

        lax.fori_loop(1, (_MCPS - 3) // 3 + 1, body, 0)
        _swait(0)
        _cpidx(_MCPS - 2, 0)
        _gather(_MCPS - 2, 0)
        _gwait(1)
        _scat(1)
        _swait(1)
        _cpidx(_MCPS - 1, 1)
        _gather(_MCPS - 1, 1)
        _gwait(2)
        _scat(2)
        _gwait(0)
        _scat(0)
        _gwait(1)
        _scat(1)
        _swait(2)
        _swait(0)
        _swait(1)

    plsc.subcore_barrier()
    pltpu.sync_copy(acc_sh.at[pl.ds(sid * _STRIPE, _STRIPE)],
                    out_hbm.at[cid, pl.ds(sid * _STRIPE, _STRIPE)])

    @pl.when(sid == _NS - 1)
    def _():
        pltpu.sync_copy(acc_sh.at[pl.ds(_TAIL_OFF, _TAIL)],
                        out_hbm.at[cid, pl.ds(_TAIL_OFF, _TAIL)])


_mp = pl.kernel(
    _mp_body,
    out_type=jax.ShapeDtypeStruct((_NC, _N, _H), jnp.float32),
    mesh=_MESH,
    scratch_types=[
        pltpu.VMEM_SHARED((_N, _H), jnp.float32),
        pltpu.VMEM((_MST,), jnp.int32),
        pltpu.VMEM((_MST,), jnp.int32),
        pltpu.VMEM((_MST,), jnp.int32),
        pltpu.VMEM((_MST,), jnp.int32),
        pltpu.VMEM((_CH,), jnp.int32),
        pltpu.VMEM((_CH,), jnp.int32),
        pltpu.VMEM((_CH,), jnp.int32),
        pltpu.VMEM((_CH, _H), jnp.float32),
        pltpu.VMEM((_CH, _H), jnp.float32),
        pltpu.VMEM((_CH, _H), jnp.float32),
        pltpu.SemaphoreType.DMA,
        pltpu.SemaphoreType.DMA,
        pltpu.SemaphoreType.DMA,
        pltpu.SemaphoreType.DMA,
        pltpu.SemaphoreType.DMA,
        pltpu.SemaphoreType.DMA,
        pltpu.SemaphoreType.DMA,
        pltpu.SemaphoreType.DMA,
    ],
)


def _emb_body(nt_ref, feats_ref, degp_ref, w1a_ref, w2b_ref, bc_ref,
              u0_ref, dinv_ref):
    oh = (nt_ref[:] == lax.broadcasted_iota(jnp.int32, (1, _NT), 1))
    h0 = (jnp.dot(oh.astype(jnp.float32), w1a_ref[:],
                  preferred_element_type=jnp.float32)
          + jnp.dot(feats_ref[:], w2b_ref[:],
                    preferred_element_type=jnp.float32) + bc_ref[:])
    deg = degp_ref[0, :, 0:1] + degp_ref[1, :, 0:1] - 1.0
    dinv = lax.rsqrt(deg)
    u0_ref[:] = h0 * dinv
    dinv_ref[:] = dinv


_emb = pl.pallas_call(
    _emb_body,
    out_shape=[
        jax.ShapeDtypeStruct((_N, _H), jnp.float32),
        jax.ShapeDtypeStruct((_N, 1), jnp.float32),
    ],
)


def _comb_body(accp_ref, u_ref, dinv_ref, b_ref, gam_ref, bet_ref, w_ref,
               out_ref):
    s = accp_ref[0] + accp_ref[1] - u_ref[:]
    t = s * dinv_ref[:] + b_ref[:]
    mu = jnp.mean(t, axis=0, keepdims=True)
    var = jnp.mean((t - mu) * (t - mu), axis=0, keepdims=True)
    zn = (t - mu) * lax.rsqrt(var + 1e-5) * gam_ref[:] + bet_ref[:]
    z = jnp.maximum(zn, 0.0)
    out_ref[:] = jnp.dot(z, w_ref[:],
                         preferred_element_type=jnp.float32) * dinv_ref[:]


_comb = pl.pallas_call(
    _comb_body,
    out_shape=jax.ShapeDtypeStruct((_N, _H), jnp.float32),
)


def _final_body(accp_ref, u_ref, dinv_ref, b_ref, batch_ref, wo_ref, bo_ref,
                out_ref):
    s = accp_ref[0] + accp_ref[1] - u_ref[:]
    t = s * dinv_ref[:] + b_ref[:]
    z = jnp.maximum(t, 0.0)
    ohb = (batch_ref[:] == lax.broadcasted_iota(jnp.int32, (1, _G), 1))
    zz = jnp.concatenate([z, jnp.ones((_N, 1), jnp.float32)], axis=1)
    s2 = lax.dot_general(ohb.astype(jnp.float32), zz,
                         (((0,), (0,)), ((), ())),
                         preferred_element_type=jnp.float32)
    pooled = s2[:, :_H] / jnp.maximum(s2[:, _H:_H + 1], 1.0)
    out_ref[:] = jnp.dot(pooled, wo_ref[:],
                         preferred_element_type=jnp.float32) + bo_ref[:]


_final = pl.pallas_call(
    _final_body,
    out_shape=jax.ShapeDtypeStruct((_G, 4), jnp.float32),
)


def kernel(node_type, c, gm, pos, r, vid, edge_index, batch, W1, b1, W2, b2,
           Wg0, bg0, Wg1, bg1, Wg2, bg2, gam0, bet0, gam1, bet1, Wo, bo):
    nt2 = node_type.reshape(_N, 1).astype(jnp.int32)
    feats = jnp.stack([c, gm, pos, r, vid], axis=-1)
    srcr = edge_index[0].astype(jnp.int32).reshape(_E)
    dstr = edge_index[1].astype(jnp.int32).reshape(_E)
    batch2 = batch.reshape(_N, 1).astype(jnp.int32)
    ones16 = jnp.ones((_N, 16), jnp.float32)

    degp = _deg(dstr, ones16)
    a0, a1 = Wg0[:_H], Wg0[_H:]
    w1a = W1 @ a0
    w2b = W2 @ a1
    bc = (b1 @ a0 + b2 @ a1).reshape(1, _H)
    u0, dinv = _emb(nt2, feats, degp, w1a, w2b, bc)
    acc0 = _mp(u0, srcr, dstr)
    u1 = _comb(acc0, u0, dinv, bg0.reshape(1, _H), gam0.reshape(1, _H),
               bet0.reshape(1, _H), Wg1)
    acc1 = _mp(u1, srcr, dstr)
    u2 = _comb(acc1, u1, dinv, bg1.reshape(1, _H), gam1.reshape(1, _H),
               bet1.reshape(1, _H), Wg2)
    acc2 = _mp(u2, srcr, dstr)
    return _final(acc2, u2, dinv, bg2.reshape(1, _H), batch2, Wo,
                  bo.reshape(1, 4))
